# Initial kernel scaffold; baseline (speedup 1.0000x reference)
import functools

import jax
import jax.numpy as jnp
from jax import lax
from jax.experimental import pallas as pl
from jax.experimental.pallas import tpu as pltpu
from jax.experimental.pallas import tpu_sc as plsc

NC = 2
NS = 16
CHUNK = 128
N_PAD = 10240
BN = 1024
NBLK = N_PAD // BN


def _mesh():
    return plsc.VectorSubcoreMesh(core_axis_name="c", subcore_axis_name="s")


def _make_sc_deg(e_pad):
    ept = e_pad // (NC * NS)
    n_chunks = ept // CHUNK
    rpt = N_PAD // NS

    @functools.partial(
        pl.kernel,
        out_type=jax.ShapeDtypeStruct((NC * N_PAD, 16), jnp.float32),
        mesh=_mesh(),
        scratch_types=[
            pltpu.VMEM((CHUNK,), jnp.int32),
            pltpu.VMEM((CHUNK, 16), jnp.float32),
            pltpu.VMEM((rpt, 16), jnp.float32),
            pltpu.VMEM_SHARED((N_PAD, 16), jnp.float32),
        ],
    )
    def k(dst_hbm, out_hbm, dst_v, ones_v, zrow_v, acc):
        c = lax.axis_index("c")
        s = lax.axis_index("s")
        wid = c * NS + s

        def fill_ones(i, _):
            ones_v[i, :] = jnp.ones((16,), jnp.float32)
            return 0

        lax.fori_loop(0, CHUNK, fill_ones, 0)

        def fill_zero(i, _):
            zrow_v[i, :] = jnp.zeros((16,), jnp.float32)
            return 0

        lax.fori_loop(0, rpt, fill_zero, 0)
        pltpu.sync_copy(zrow_v, acc.at[pl.ds(s * rpt, rpt)])
        plsc.subcore_barrier()

        def body(i, _):
            base = wid * ept + i * CHUNK
            pltpu.sync_copy(dst_hbm.at[pl.ds(base, CHUNK)], dst_v)
            pltpu.sync_copy(ones_v, acc.at[dst_v], add=True)
            return 0

        lax.fori_loop(0, n_chunks, body, 0)
        plsc.subcore_barrier()
        pltpu.sync_copy(acc.at[pl.ds(s * rpt, rpt)],
                        out_hbm.at[pl.ds(c * N_PAD + s * rpt, rpt)])

    return k


def _make_sc_agg(w2, e_pad):
    ept = e_pad // NS
    n_chunks = ept // CHUNK
    rpt = N_PAD // NS

    @functools.partial(
        pl.kernel,
        out_type=jax.ShapeDtypeStruct((NC * N_PAD, w2), jnp.float32),
        mesh=_mesh(),
        scratch_types=[
            pltpu.VMEM((CHUNK,), jnp.int32),
            pltpu.VMEM((CHUNK,), jnp.int32),
            pltpu.VMEM((CHUNK, w2), jnp.float32),
            pltpu.VMEM_SHARED((N_PAD, w2), jnp.float32),
            pltpu.SemaphoreType.DMA,
        ],
    )
    def k(y_hbm, src_hbm, dst_hbm, out_hbm, src_v, dst_v, rows_v, acc, sem):
        c = lax.axis_index("c")
        s = lax.axis_index("s")
        row0 = c * N_PAD
        pltpu.sync_copy(y_hbm.at[pl.ds(row0 + s * rpt, rpt)],
                        acc.at[pl.ds(s * rpt, rpt)])
        plsc.subcore_barrier()

        def body(i, _):
            base = s * ept + i * CHUNK
            pltpu.sync_copy(src_hbm.at[pl.ds(base, CHUNK)], src_v)
            pltpu.sync_copy(dst_hbm.at[pl.ds(base, CHUNK)], dst_v)
            for j in range(CHUNK // 16):
                sl = pl.ds(j * 16, 16)
                src_v[sl] = src_v[sl] + row0
            pltpu.async_copy(y_hbm.at[src_v], rows_v, sem).wait()
            pltpu.sync_copy(rows_v, acc.at[dst_v], add=True)
            return 0

        lax.fori_loop(0, n_chunks, body, 0)
        plsc.subcore_barrier()
        pltpu.sync_copy(acc.at[pl.ds(s * rpt, rpt)],
                        out_hbm.at[pl.ds(row0 + s * rpt, rpt)])

    return k


def _dinv_body(p0_ref, p1_ref, out_ref):
    cnt = p0_ref[:, 0:1] + p1_ref[:, 0:1]
    out_ref[...] = lax.rsqrt(cnt + 1.0)


def _t_dinv(partials):
    return pl.pallas_call(
        _dinv_body,
        grid=(NBLK,),
        in_specs=[
            pl.BlockSpec((BN, 16), lambda i: (i, 0)),
            pl.BlockSpec((BN, 16), lambda i: (N_PAD // BN + i, 0)),
        ],
        out_specs=pl.BlockSpec((BN, 1), lambda i: (i, 0)),
        out_shape=jax.ShapeDtypeStruct((N_PAD, 1), jnp.float32),
    )(partials, partials)


def _lin1_body(x_ref, w_ref, dinv_ref, out_ref):
    out_ref[...] = dinv_ref[...] * jnp.dot(
        x_ref[...], w_ref[...], preferred_element_type=jnp.float32)


def _t_lin1(x_p, W1, dinv):
    w2 = W1.shape[1] // 2
    return pl.pallas_call(
        _lin1_body,
        grid=(NC, NBLK),
        in_specs=[
            pl.BlockSpec((BN, W1.shape[0]), lambda c, i: (i, 0)),
            pl.BlockSpec((W1.shape[0], w2), lambda c, i: (0, c)),
            pl.BlockSpec((BN, 1), lambda c, i: (i, 0)),
        ],
        out_specs=pl.BlockSpec((BN, w2), lambda c, i: (c * NBLK + i, 0)),
        out_shape=jax.ShapeDtypeStruct((NC * N_PAD, w2), jnp.float32),
    )(x_p, W1, dinv)


def _layer_body(a0_ref, a1_ref, b_ref, w_ref, dinv_ref, out_ref):
    h = jnp.concatenate([a0_ref[...], a1_ref[...]], axis=1)
    h = jnp.maximum(dinv_ref[...] * h + b_ref[...], 0.0)
    out_ref[...] = dinv_ref[...] * jnp.dot(
        h, w_ref[...], preferred_element_type=jnp.float32)


def _t_layer(acc, b_prev, W, dinv):
    w_in = W.shape[0]
    w2_in = w_in // 2
    w2 = W.shape[1] // 2
    return pl.pallas_call(
        _layer_body,
        grid=(NC, NBLK),
        in_specs=[
            pl.BlockSpec((BN, w2_in), lambda c, i: (i, 0)),
            pl.BlockSpec((BN, w2_in), lambda c, i: (NBLK + i, 0)),
            pl.BlockSpec((1, w_in), lambda c, i: (0, 0)),
            pl.BlockSpec((w_in, w2), lambda c, i: (0, c)),
            pl.BlockSpec((BN, 1), lambda c, i: (i, 0)),
        ],
        out_specs=pl.BlockSpec((BN, w2), lambda c, i: (c * NBLK + i, 0)),
        out_shape=jax.ShapeDtypeStruct((NC * N_PAD, w2), jnp.float32),
    )(acc, acc, b_prev.reshape(1, -1), W, dinv)


def _head_body(a0_ref, a1_ref, b4_ref, w_ref, b5_ref, dinv_ref, out_ref):
    h = jnp.concatenate([a0_ref[...], a1_ref[...]], axis=1)
    h = jnp.maximum(dinv_ref[...] * h + b4_ref[...], 0.0)
    out_ref[...] = jnp.dot(
        h, w_ref[...], preferred_element_type=jnp.float32) + b5_ref[...]


def _t_head(acc, b4, W5, b5, dinv):
    w_in = W5.shape[0]
    w2_in = w_in // 2
    c_out = W5.shape[1]
    return pl.pallas_call(
        _head_body,
        grid=(NBLK,),
        in_specs=[
            pl.BlockSpec((BN, w2_in), lambda i: (i, 0)),
            pl.BlockSpec((BN, w2_in), lambda i: (NBLK + i, 0)),
            pl.BlockSpec((1, w_in), lambda i: (0, 0)),
            pl.BlockSpec((w_in, c_out), lambda i: (0, 0)),
            pl.BlockSpec((1, c_out), lambda i: (0, 0)),
            pl.BlockSpec((BN, 1), lambda i: (i, 0)),
        ],
        out_specs=pl.BlockSpec((BN, c_out), lambda i: (i, 0)),
        out_shape=jax.ShapeDtypeStruct((N_PAD, c_out), jnp.float32),
    )(acc, acc, b4.reshape(1, -1), W5, b5.reshape(1, -1), dinv)


def kernel(x, edge_index, batch, W1, b1, W2, b2, W3, b3, W4, b4, W5, b5):
    n = x.shape[0]
    e = edge_index.shape[1]
    e_pad = -(-e // (NC * NS * CHUNK)) * (NC * NS * CHUNK)

    src = edge_index[0].astype(jnp.int32)
    dst = edge_index[1].astype(jnp.int32)
    pad = e_pad - e
    src_p = jnp.concatenate([src, jnp.zeros((pad,), jnp.int32)])
    dst_p = jnp.concatenate([dst, jnp.full((pad,), n, jnp.int32)])
    x_p = jnp.pad(x, ((0, N_PAD - n), (0, 0)))

    sc_deg = _make_sc_deg(e_pad)
    sc_agg64 = _make_sc_agg(64, e_pad)
    sc_agg128 = _make_sc_agg(128, e_pad)

    deg_part = sc_deg(dst_p)
    dinv = _t_dinv(deg_part)

    y1 = _t_lin1(x_p, W1, dinv)
    a1 = sc_agg64(y1, src_p, dst_p)
    y2 = _t_layer(a1, b1, W2, dinv)
    a2 = sc_agg128(y2, src_p, dst_p)
    y3 = _t_layer(a2, b2, W3, dinv)
    a3 = sc_agg128(y3, src_p, dst_p)
    y4 = _t_layer(a3, b3, W4, dinv)
    a4 = sc_agg64(y4, src_p, dst_p)
    out = _t_head(a4, b4, W5, b5, dinv)
    return out[:n]


# speedup vs baseline: 7.3332x; 7.3332x over previous
import functools

import jax
import jax.numpy as jnp
from jax import lax
from jax.experimental import pallas as pl
from jax.experimental.pallas import tpu as pltpu
from jax.experimental.pallas import tpu_sc as plsc

NC = 2
NS = 16
CHUNK = 128
W2 = 128
N_PAD = 10240
BN = 1024
NBLK = N_PAD // BN
RPT = N_PAD // NS


def _mesh():
    return plsc.VectorSubcoreMesh(core_axis_name="c", subcore_axis_name="s")


def _make_sc_deg(e_pad):
    ept = e_pad // (NC * NS)
    n_chunks = ept // CHUNK

    @functools.partial(
        pl.kernel,
        out_type=jax.ShapeDtypeStruct((NC * N_PAD, W2), jnp.float32),
        mesh=_mesh(),
        scratch_types=[
            pltpu.VMEM((CHUNK,), jnp.int32),
            pltpu.VMEM((CHUNK, W2), jnp.float32),
            pltpu.VMEM_SHARED((N_PAD, W2), jnp.float32),
        ],
    )
    def k(dst_hbm, out_hbm, dst_v, ones_v, acc):
        c = lax.axis_index("c")
        s = lax.axis_index("s")
        wid = c * NS + s

        def fill_zero(i, _):
            for j in range(W2 // 16):
                ones_v[i, pl.ds(j * 16, 16)] = jnp.zeros((16,), jnp.float32)
            return 0

        lax.fori_loop(0, CHUNK, fill_zero, 0)

        def zero_acc(i, _):
            pltpu.sync_copy(ones_v, acc.at[pl.ds(s * RPT + i * CHUNK, CHUNK)])
            return 0

        lax.fori_loop(0, RPT // CHUNK, zero_acc, 0)

        def fill_ones(i, _):
            for j in range(W2 // 16):
                ones_v[i, pl.ds(j * 16, 16)] = jnp.ones((16,), jnp.float32)
            return 0

        lax.fori_loop(0, CHUNK, fill_ones, 0)
        plsc.subcore_barrier()

        def body(i, _):
            base = wid * ept + i * CHUNK
            pltpu.sync_copy(dst_hbm.at[pl.ds(base, CHUNK)], dst_v)
            pltpu.sync_copy(ones_v, acc.at[dst_v], add=True)
            return 0

        lax.fori_loop(0, n_chunks, body, 0)
        plsc.subcore_barrier()
        pltpu.sync_copy(acc.at[pl.ds(s * RPT, RPT)],
                        out_hbm.at[pl.ds(c * N_PAD + s * RPT, RPT)])

    return k


def _make_sc_agg(e_pad, feat_split):
    ept = e_pad // NS if feat_split else e_pad // (NC * NS)
    n_chunks = ept // CHUNK

    @functools.partial(
        pl.kernel,
        out_type=jax.ShapeDtypeStruct((NC * N_PAD, W2), jnp.float32),
        mesh=_mesh(),
        scratch_types=[
            pltpu.VMEM((CHUNK,), jnp.int32),
            pltpu.VMEM((CHUNK,), jnp.int32),
            pltpu.VMEM((CHUNK, W2), jnp.float32),
            pltpu.VMEM_SHARED((N_PAD, W2), jnp.float32),
            pltpu.SemaphoreType.DMA,
        ],
    )
    def k(y_hbm, src_hbm, dst_hbm, out_hbm, src_v, dst_v, rows_v, acc, sem):
        c = lax.axis_index("c")
        s = lax.axis_index("s")
        row0 = c * N_PAD
        pltpu.sync_copy(y_hbm.at[pl.ds(row0 + s * RPT, RPT)],
                        acc.at[pl.ds(s * RPT, RPT)])
        plsc.subcore_barrier()

        def body(i, _):
            if feat_split:
                base = s * ept + i * CHUNK
            else:
                base = (c * NS + s) * ept + i * CHUNK
            pltpu.sync_copy(src_hbm.at[pl.ds(base, CHUNK)], src_v)
            pltpu.sync_copy(dst_hbm.at[pl.ds(base, CHUNK)], dst_v)
            if feat_split:
                for j in range(CHUNK // 16):
                    sl = pl.ds(j * 16, 16)
                    src_v[sl] = src_v[sl] + row0
            pltpu.async_copy(y_hbm.at[src_v], rows_v, sem).wait()
            pltpu.sync_copy(rows_v, acc.at[dst_v], add=True)
            return 0

        lax.fori_loop(0, n_chunks, body, 0)
        plsc.subcore_barrier()
        pltpu.sync_copy(acc.at[pl.ds(s * RPT, RPT)],
                        out_hbm.at[pl.ds(row0 + s * RPT, RPT)])

    return k


def _split_w(W):
    w_in, w_out = W.shape
    return jnp.moveaxis(W.reshape(w_in, 2, w_out // 2), 1, 0)


def _dinv_body(p0_ref, p1_ref, out_ref):
    cnt = p0_ref[:, 0:1] + p1_ref[:, 0:1]
    out_ref[...] = lax.rsqrt(cnt + 1.0)


def _t_dinv(partials):
    return pl.pallas_call(
        _dinv_body,
        grid=(NBLK,),
        in_specs=[
            pl.BlockSpec((BN, W2), lambda i: (i, 0)),
            pl.BlockSpec((BN, W2), lambda i: (NBLK + i, 0)),
        ],
        out_specs=pl.BlockSpec((BN, 1), lambda i: (i, 0)),
        out_shape=jax.ShapeDtypeStruct((N_PAD, 1), jnp.float32),
    )(partials, partials)


def _lin1_body(x_ref, w_ref, dinv_ref, out_ref):
    c = pl.program_id(0)

    @pl.when(c == 0)
    def _():
        out_ref[...] = dinv_ref[...] * jnp.dot(
            x_ref[...], w_ref[...], preferred_element_type=jnp.float32)

    @pl.when(c != 0)
    def _():
        out_ref[...] = jnp.zeros_like(out_ref)


def _t_lin1(x_p, W1, dinv):
    w_in, w_out = W1.shape
    return pl.pallas_call(
        _lin1_body,
        grid=(NC, NBLK),
        in_specs=[
            pl.BlockSpec((BN, w_in), lambda c, i: (i, 0)),
            pl.BlockSpec((w_in, w_out), lambda c, i: (0, 0)),
            pl.BlockSpec((BN, 1), lambda c, i: (i, 0)),
        ],
        out_specs=pl.BlockSpec((BN, w_out), lambda c, i: (c * NBLK + i, 0)),
        out_shape=jax.ShapeDtypeStruct((NC * N_PAD, w_out), jnp.float32),
    )(x_p, W1, dinv)


def _sum_to_feat_body(p0_ref, p1_ref, b_ref, w_ref, dinv_ref, out_ref):
    h = p0_ref[...] + p1_ref[...]
    h = jnp.maximum(dinv_ref[...] * h + b_ref[...], 0.0)
    out_ref[...] = dinv_ref[...] * jnp.dot(
        h, w_ref[0], preferred_element_type=jnp.float32)


def _t_sum_to_feat(acc, b_prev, W, dinv):
    w_in = W.shape[0]
    w2 = W.shape[1] // 2
    return pl.pallas_call(
        _sum_to_feat_body,
        grid=(NC, NBLK),
        in_specs=[
            pl.BlockSpec((BN, w_in), lambda c, i: (i, 0)),
            pl.BlockSpec((BN, w_in), lambda c, i: (NBLK + i, 0)),
            pl.BlockSpec((1, w_in), lambda c, i: (0, 0)),
            pl.BlockSpec((1, w_in, w2), lambda c, i: (c, 0, 0)),
            pl.BlockSpec((BN, 1), lambda c, i: (i, 0)),
        ],
        out_specs=pl.BlockSpec((BN, w2), lambda c, i: (c * NBLK + i, 0)),
        out_shape=jax.ShapeDtypeStruct((NC * N_PAD, w2), jnp.float32),
    )(acc, acc, b_prev.reshape(1, -1), _split_w(W), dinv)


def _feat_to_feat_body(a0_ref, a1_ref, b_ref, w_ref, dinv_ref, out_ref):
    h = jnp.concatenate([a0_ref[...], a1_ref[...]], axis=1)
    h = jnp.maximum(dinv_ref[...] * h + b_ref[...], 0.0)
    out_ref[...] = dinv_ref[...] * jnp.dot(
        h, w_ref[0], preferred_element_type=jnp.float32)


def _t_feat_to_feat(acc, b_prev, W, dinv):
    w_in = W.shape[0]
    w2_in = w_in // 2
    w2 = W.shape[1] // 2
    return pl.pallas_call(
        _feat_to_feat_body,
        grid=(NC, NBLK),
        in_specs=[
            pl.BlockSpec((BN, w2_in), lambda c, i: (i, 0)),
            pl.BlockSpec((BN, w2_in), lambda c, i: (NBLK + i, 0)),
            pl.BlockSpec((1, w_in), lambda c, i: (0, 0)),
            pl.BlockSpec((1, w_in, w2), lambda c, i: (c, 0, 0)),
            pl.BlockSpec((BN, 1), lambda c, i: (i, 0)),
        ],
        out_specs=pl.BlockSpec((BN, w2), lambda c, i: (c * NBLK + i, 0)),
        out_shape=jax.ShapeDtypeStruct((NC * N_PAD, w2), jnp.float32),
    )(acc, acc, b_prev.reshape(1, -1), _split_w(W), dinv)


def _feat_to_edge_body(a0_ref, a1_ref, b_ref, w_ref, dinv_ref, out_ref):
    c = pl.program_id(0)

    @pl.when(c == 0)
    def _():
        h = jnp.concatenate([a0_ref[...], a1_ref[...]], axis=1)
        h = jnp.maximum(dinv_ref[...] * h + b_ref[...], 0.0)
        out_ref[...] = dinv_ref[...] * jnp.dot(
            h, w_ref[...], preferred_element_type=jnp.float32)

    @pl.when(c != 0)
    def _():
        out_ref[...] = jnp.zeros_like(out_ref)


def _t_feat_to_edge(acc, b_prev, W, dinv):
    w_in, w_out = W.shape
    w2_in = w_in // 2
    return pl.pallas_call(
        _feat_to_edge_body,
        grid=(NC, NBLK),
        in_specs=[
            pl.BlockSpec((BN, w2_in), lambda c, i: (i, 0)),
            pl.BlockSpec((BN, w2_in), lambda c, i: (NBLK + i, 0)),
            pl.BlockSpec((1, w_in), lambda c, i: (0, 0)),
            pl.BlockSpec((w_in, w_out), lambda c, i: (0, 0)),
            pl.BlockSpec((BN, 1), lambda c, i: (i, 0)),
        ],
        out_specs=pl.BlockSpec((BN, w_out), lambda c, i: (c * NBLK + i, 0)),
        out_shape=jax.ShapeDtypeStruct((NC * N_PAD, w_out), jnp.float32),
    )(acc, acc, b_prev.reshape(1, -1), W, dinv)


def _head_body(p0_ref, p1_ref, b4_ref, w_ref, b5_ref, dinv_ref, out_ref):
    h = p0_ref[...] + p1_ref[...]
    h = jnp.maximum(dinv_ref[...] * h + b4_ref[...], 0.0)
    out_ref[...] = jnp.dot(
        h, w_ref[...], preferred_element_type=jnp.float32) + b5_ref[...]


def _t_head(acc, b4, W5, b5, dinv):
    w_in, c_out = W5.shape
    return pl.pallas_call(
        _head_body,
        grid=(NBLK,),
        in_specs=[
            pl.BlockSpec((BN, w_in), lambda i: (i, 0)),
            pl.BlockSpec((BN, w_in), lambda i: (NBLK + i, 0)),
            pl.BlockSpec((1, w_in), lambda i: (0, 0)),
            pl.BlockSpec((w_in, c_out), lambda i: (0, 0)),
            pl.BlockSpec((1, c_out), lambda i: (0, 0)),
            pl.BlockSpec((BN, 1), lambda i: (i, 0)),
        ],
        out_specs=pl.BlockSpec((BN, c_out), lambda i: (i, 0)),
        out_shape=jax.ShapeDtypeStruct((N_PAD, c_out), jnp.float32),
    )(acc, acc, b4.reshape(1, -1), W5, b5.reshape(1, -1), dinv)


def kernel(x, edge_index, batch, W1, b1, W2_, b2, W3, b3, W4, b4, W5, b5):
    n = x.shape[0]
    e = edge_index.shape[1]
    e_pad = -(-e // (NC * NS * CHUNK)) * (NC * NS * CHUNK)

    src = edge_index[0].astype(jnp.int32)
    dst = edge_index[1].astype(jnp.int32)
    pad = e_pad - e
    src_p = jnp.concatenate([src, jnp.zeros((pad,), jnp.int32)])
    dst_p = jnp.concatenate([dst, jnp.full((pad,), n, jnp.int32)])
    x_p = jnp.pad(x, ((0, N_PAD - n), (0, 0)))

    sc_deg = _make_sc_deg(e_pad)
    sc_agg_edge = _make_sc_agg(e_pad, feat_split=False)
    sc_agg_feat = _make_sc_agg(e_pad, feat_split=True)

    deg_part = sc_deg(dst_p)
    dinv = _t_dinv(deg_part)

    y1 = _t_lin1(x_p, W1, dinv)
    a1 = sc_agg_edge(y1, src_p, dst_p)
    y2 = _t_sum_to_feat(a1, b1, W2_, dinv)
    a2 = sc_agg_feat(y2, src_p, dst_p)
    y3 = _t_feat_to_feat(a2, b2, W3, dinv)
    a3 = sc_agg_feat(y3, src_p, dst_p)
    y4 = _t_feat_to_edge(a3, b3, W4, dinv)
    a4 = sc_agg_edge(y4, src_p, dst_p)
    out = _t_head(a4, b4, W5, b5, dinv)
    return out[:n]
